# baseline (speedup 1.0000x reference)
import jax
import jax.numpy as jnp
from jax.experimental import pallas as pl

GRID_SIZE = 0.01
SCALE_MULTIPLIER = 0.04
RADII_MIN, RADII_MAX = 1, 18

_TN = 2000


def _agg_body(pts_ref, g_ref, sem_ref, logit_ref, bin_ref, dens_ref):
    gx = g_ref[0:1, :]
    gy = g_ref[1:2, :]
    gz = g_ref[2:3, :]
    sx = g_ref[3:4, :]
    sy = g_ref[4:5, :]
    sz = g_ref[5:6, :]
    opa = g_ref[6:7, :]
    c0 = g_ref[7:8, :]
    c1 = g_ref[8:9, :]
    c2 = g_ref[9:10, :]
    c3 = g_ref[10:11, :]
    c4 = g_ref[11:12, :]
    c5 = g_ref[12:13, :]

    mix = ((gx - 0.0) / GRID_SIZE).astype(jnp.int32)
    miy = ((gy - 0.0) / GRID_SIZE).astype(jnp.int32)
    miz = ((gz - 0.0) / GRID_SIZE).astype(jnp.int32)
    rmax = jnp.maximum(jnp.maximum(sx, sy), sz)
    radii = jnp.ceil(rmax * SCALE_MULTIPLIER / GRID_SIZE).astype(jnp.int32)
    radii = jnp.clip(radii, RADII_MIN, RADII_MAX)
    h0 = -0.5 * c0
    h1 = -0.5 * c1
    h2 = -0.5 * c2
    sem = sem_ref[...]

    _H = (512 << 20) | (512 << 10) | 512
    glo = (((mix - radii + 64) << 20)
           | ((miy - radii + 64) << 10)
           | (miz - radii + 64))
    ghiH = ((((mix + radii + 64) << 20)
             | ((miy + radii + 64) << 10)
             | (miz + radii + 64)) + _H)

    nsub = pts_ref.shape[0] // _TN

    def step(j, _):
        sl = pl.ds(j * _TN, _TN)
        px = pts_ref[sl, 0:1]
        py = pts_ref[sl, 1:2]
        pz = pts_ref[sl, 2:3]
        pix = ((px - 0.0) / GRID_SIZE).astype(jnp.int32)
        piy = ((py - 0.0) / GRID_SIZE).astype(jnp.int32)
        piz = ((pz - 0.0) / GRID_SIZE).astype(jnp.int32)
        pcode = (((pix + 64) << 20) | ((piy + 64) << 10) | (piz + 64))
        pH = pcode + _H

        dx = px - gx
        dy = py - gy
        dz = pz - gz
        power = (dx * (h0 * dx - c3 * dy - c5 * dz)
                 + dy * (h1 * dy - c4 * dz)
                 + h2 * dz * dz)
        in_range = ((pH - glo) & (ghiH - pcode) & _H) == _H
        valid = in_range & (power <= 0.0)
        a = jnp.where(valid, jnp.exp(power) * opa, 0.0)

        density = jnp.sum(a, axis=1, keepdims=True)
        m1 = 1.0 - a
        q = m1[:, 0:128]
        for k in range(1, m1.shape[1] // 128):
            q = q * m1[:, k * 128:(k + 1) * 128]
        bin_logits = 1.0 - jnp.exp(jnp.sum(jnp.log(q), axis=1, keepdims=True))
        logits = jnp.dot(a, sem, preferred_element_type=jnp.float32)

        logit_ref[sl, :] = logits
        bin_ref[sl, :] = bin_logits
        dens_ref[sl, :] = density
        return 0

    jax.lax.fori_loop(0, nsub, step, 0)


def kernel(pts, means3D, opas, semantics, scales, cov3D):
    pts = pts[0]
    means3D = means3D[0]
    opas = opas[0]
    semantics = semantics[0]
    scales = scales[0]
    cov3D = cov3D[0]
    N = pts.shape[0]
    M = means3D.shape[0]
    cov6 = cov3D.reshape(M, 9)[:, jnp.array([0, 4, 8, 1, 5, 2])]
    g = jnp.concatenate([means3D, scales, opas[:, None], cov6], axis=1).T

    out = pl.pallas_call(
        _agg_body,
        grid=(1,),
        in_specs=[
            pl.BlockSpec((N, 3), lambda i: (0, 0)),
            pl.BlockSpec((13, M), lambda i: (0, 0)),
            pl.BlockSpec((M, 17), lambda i: (0, 0)),
        ],
        out_specs=[
            pl.BlockSpec((N, 17), lambda i: (0, 0)),
            pl.BlockSpec((N, 1), lambda i: (0, 0)),
            pl.BlockSpec((N, 1), lambda i: (0, 0)),
        ],
        out_shape=[
            jax.ShapeDtypeStruct((N, 17), jnp.float32),
            jax.ShapeDtypeStruct((N, 1), jnp.float32),
            jax.ShapeDtypeStruct((N, 1), jnp.float32),
        ],
    )(pts, g, semantics)

    return out[0], out[1].reshape(N), out[2].reshape(N)


# speedup vs baseline: 1.0422x; 1.0422x over previous
import jax
import jax.numpy as jnp
from jax.experimental import pallas as pl

GRID_SIZE = 0.01
SCALE_MULTIPLIER = 0.04
RADII_MIN, RADII_MAX = 1, 18

_TN = 2000


def _agg_body(pts_ref, g_ref, sem_ref, out_ref):
    gx = g_ref[0:1, :]
    gy = g_ref[1:2, :]
    gz = g_ref[2:3, :]
    sx = g_ref[3:4, :]
    sy = g_ref[4:5, :]
    sz = g_ref[5:6, :]
    opa = g_ref[6:7, :]
    c0 = g_ref[7:8, :]
    c1 = g_ref[8:9, :]
    c2 = g_ref[9:10, :]
    c3 = g_ref[10:11, :]
    c4 = g_ref[11:12, :]
    c5 = g_ref[12:13, :]

    mix = ((gx - 0.0) / GRID_SIZE).astype(jnp.int32)
    miy = ((gy - 0.0) / GRID_SIZE).astype(jnp.int32)
    miz = ((gz - 0.0) / GRID_SIZE).astype(jnp.int32)
    rmax = jnp.maximum(jnp.maximum(sx, sy), sz)
    radii = jnp.ceil(rmax * SCALE_MULTIPLIER / GRID_SIZE).astype(jnp.int32)
    radii = jnp.clip(radii, RADII_MIN, RADII_MAX)
    h0 = -0.5 * c0
    h1 = -0.5 * c1
    h2 = -0.5 * c2
    sem = sem_ref[...]

    _H = (512 << 20) | (512 << 10) | 512
    glo = (((mix - radii + 64) << 20)
           | ((miy - radii + 64) << 10)
           | (miz - radii + 64))
    ghiH = ((((mix + radii + 64) << 20)
             | ((miy + radii + 64) << 10)
             | (miz + radii + 64)) + _H)

    nsub = pts_ref.shape[0] // _TN

    def step(j, _):
        sl = pl.ds(j * _TN, _TN)
        px = pts_ref[sl, 0:1]
        py = pts_ref[sl, 1:2]
        pz = pts_ref[sl, 2:3]
        pix = ((px - 0.0) / GRID_SIZE).astype(jnp.int32)
        piy = ((py - 0.0) / GRID_SIZE).astype(jnp.int32)
        piz = ((pz - 0.0) / GRID_SIZE).astype(jnp.int32)
        pcode = (((pix + 64) << 20) | ((piy + 64) << 10) | (piz + 64))
        pH = pcode + _H

        dx = px - gx
        dy = py - gy
        dz = pz - gz
        power = (dx * (h0 * dx - c3 * dy - c5 * dz)
                 + dy * (h1 * dy - c4 * dz)
                 + h2 * dz * dz)
        in_range = ((pH - glo) & (ghiH - pcode) & _H) == _H
        valid = in_range & (power <= 0.0)
        a = jnp.where(valid, jnp.exp(power) * opa, 0.0)

        m1 = 1.0 - a
        q = m1[:, 0:128]
        for k in range(1, m1.shape[1] // 128):
            q = q * m1[:, k * 128:(k + 1) * 128]
        bin_logits = 1.0 - jnp.exp(jnp.sum(jnp.log(q), axis=1, keepdims=True))
        logits = jnp.dot(a, sem, preferred_element_type=jnp.float32)

        out_ref[sl, 0:17] = logits[:, 0:17]
        out_ref[sl, 17:18] = bin_logits
        out_ref[sl, 18:19] = logits[:, 17:18]
        return 0

    jax.lax.fori_loop(0, nsub, step, 0)


def kernel(pts, means3D, opas, semantics, scales, cov3D):
    pts = pts[0]
    means3D = means3D[0]
    opas = opas[0]
    semantics = semantics[0]
    scales = scales[0]
    cov3D = cov3D[0]
    N = pts.shape[0]
    M = means3D.shape[0]
    cov6 = cov3D.reshape(M, 9)[:, jnp.array([0, 4, 8, 1, 5, 2])]
    g = jnp.concatenate([means3D, scales, opas[:, None], cov6], axis=1).T
    sem18 = jnp.concatenate([semantics, jnp.ones((M, 1), jnp.float32)], axis=1)

    out = pl.pallas_call(
        _agg_body,
        grid=(1,),
        in_specs=[
            pl.BlockSpec((N, 3), lambda i: (0, 0)),
            pl.BlockSpec((13, M), lambda i: (0, 0)),
            pl.BlockSpec((M, 18), lambda i: (0, 0)),
        ],
        out_specs=pl.BlockSpec((N, 19), lambda i: (0, 0)),
        out_shape=jax.ShapeDtypeStruct((N, 19), jnp.float32),
    )(pts, g, sem18)

    return out[:, 0:17], out[:, 17], out[:, 18]
